# Initial kernel scaffold; baseline (speedup 1.0000x reference)
import jax
import jax.numpy as jnp
from jax.experimental import pallas as pl


def kernel(hidden_states, gate_w, w_gate_up, w_down, shared_gate_up, shared_down):
    raise NotImplementedError("write your pallas kernel here")


# speedup vs baseline: 40.6371x; 40.6371x over previous
import jax
import jax.numpy as jnp
from jax.experimental import pallas as pl
from jax.experimental.pallas import tpu as pltpu

_T = 2048
_D = 1024
_E = 64
_K = 2
_FF = 512
_SFF = 512
_P = _T * _K
_TILE = 128
_NT_PAD = _P + _E * _TILE
_NUM_TILES = _NT_PAD // _TILE
_TT = 128
_CH = 1024


def _cumsum_rows(x):
    n = x.shape[0]
    sh = 1
    while sh < n:
        pad = jnp.zeros((sh, x.shape[1]), x.dtype)
        x = x + jnp.concatenate([pad, x[:-sh]], axis=0)
        sh *= 2
    return x


def _cumsum_lanes(x):
    n = x.shape[1]
    sh = 1
    while sh < n:
        pad = jnp.zeros((x.shape[0], sh), x.dtype)
        x = x + jnp.concatenate([pad, x[:, :-sh]], axis=1)
        sh *= 2
    return x


def _router_kernel(x_ref, gw_ref, w_ref, pos_ref, st_ref, te_ref):
    x = x_ref[...]
    gw = gw_ref[...]
    logits = jax.lax.dot_general(
        x, gw, (((1,), (1,)), ((), ())),
        preferred_element_type=jnp.float32)

    l1 = jnp.max(logits, axis=1, keepdims=True)
    i1 = jnp.argmax(logits, axis=1, keepdims=True)
    ecol = jax.lax.broadcasted_iota(jnp.int32, (_T, _E), 1)
    masked = jnp.where(ecol == i1, -jnp.inf, logits)
    l2 = jnp.max(masked, axis=1, keepdims=True)
    i2 = jnp.argmax(masked, axis=1, keepdims=True)
    r = jnp.exp(l2 - l1)
    s = 1.0 + r
    w_ref[...] = jnp.concatenate([1.0 / s, r / s], axis=1)

    oh = jnp.concatenate([(ecol == i1), (ecol == i2)],
                         axis=0).astype(jnp.int32)
    csum = _cumsum_rows(oh)
    counts = csum[_P - 1:_P, :]
    rank = jnp.sum(oh * csum, axis=1, keepdims=True) - 1
    pc = ((counts + (_TILE - 1)) // _TILE) * _TILE
    cpc = _cumsum_lanes(pc)
    po = cpc - pc
    pos_flat = jnp.sum(oh * po, axis=1, keepdims=True) + rank
    pos_ref[...] = jnp.concatenate([pos_flat[:_T], pos_flat[_T:]], axis=1)

    trow = jax.lax.broadcasted_iota(jnp.int32, (_NUM_TILES, _E), 0) * _TILE
    te = jnp.sum((trow >= cpc).astype(jnp.int32), axis=1, keepdims=True)
    te_ref[...] = jnp.minimum(te, _E - 1)

    tok_half = jax.lax.broadcasted_iota(jnp.int32, (_T, 1), 0)
    tok = jnp.concatenate([tok_half, tok_half], axis=0)

    def _slot_body(j, _):
        srow = jax.lax.broadcasted_iota(jnp.int32, (1, _TILE), 1) + j * _TILE
        m = (pos_flat == srow).astype(jnp.int32)
        vals = jnp.sum(m * tok, axis=0)
        st_ref[pl.ds(j * _TILE, _TILE), 0] = vals
        return 0

    jax.lax.fori_loop(0, _NUM_TILES, _slot_body, 0)


def _expert_kernel(te_ref, stok_ref, x_ref, w1_ref, w2_ref, out_ref):
    del te_ref
    tok = stok_ref[...]
    col = jax.lax.broadcasted_iota(jnp.int32, (_TILE, _T), 1)
    oh = (tok == col).astype(jnp.bfloat16)
    xg = jnp.dot(oh, x_ref[...], preferred_element_type=jnp.float32)
    h = jnp.dot(xg.astype(jnp.bfloat16), w1_ref[0].astype(jnp.bfloat16),
                preferred_element_type=jnp.float32)
    g = h[:, :_FF]
    u = h[:, _FF:]
    a = (jax.nn.silu(g) * u).astype(jnp.bfloat16)
    o = jnp.dot(a, w2_ref[0].astype(jnp.bfloat16),
                preferred_element_type=jnp.float32)
    out_ref[...] = o.astype(jnp.bfloat16)


def _combine_kernel(x_ref, sgu_ref, sd_ref, os_ref, pos_ref, w_ref, out_ref):
    h = jnp.dot(x_ref[...], sgu_ref[...], preferred_element_type=jnp.float32)
    g = h[:, :_SFF]
    u = h[:, _SFF:]
    a = (jax.nn.silu(g) * u).astype(jnp.bfloat16)
    shared = jnp.dot(a, sd_ref[...], preferred_element_type=jnp.float32)

    pos0 = pos_ref[:, 0:1]
    pos1 = pos_ref[:, 1:2]
    w0 = w_ref[:, 0:1]
    w1 = w_ref[:, 1:2]
    srow = jax.lax.broadcasted_iota(jnp.int32, (_TT, _NT_PAD), 1)
    gmat = ((pos0 == srow).astype(jnp.float32) * w0 +
            (pos1 == srow).astype(jnp.float32) * w1).astype(jnp.bfloat16)
    routed = jnp.dot(gmat, os_ref[...], preferred_element_type=jnp.float32)
    out_ref[...] = routed + shared


def kernel(hidden_states, gate_w, w_gate_up, w_down, shared_gate_up, shared_down):
    x_bf = hidden_states.astype(jnp.bfloat16)
    sgu_bf = shared_gate_up.astype(jnp.bfloat16)
    sd_bf = shared_down.astype(jnp.bfloat16)

    topk_w, pos2, stok, te = pl.pallas_call(
        _router_kernel,
        out_shape=[
            jax.ShapeDtypeStruct((_T, _K), jnp.float32),
            jax.ShapeDtypeStruct((_T, _K), jnp.int32),
            jax.ShapeDtypeStruct((_NT_PAD, 1), jnp.int32),
            jax.ShapeDtypeStruct((_NUM_TILES, 1), jnp.int32),
        ],
    )(hidden_states, gate_w)

    te_flat = te.reshape(_NUM_TILES)

    os = pl.pallas_call(
        _expert_kernel,
        grid_spec=pltpu.PrefetchScalarGridSpec(
            num_scalar_prefetch=1,
            grid=(_NUM_TILES,),
            in_specs=[
                pl.BlockSpec((_TILE, 1), lambda t, te: (t, 0)),
                pl.BlockSpec((_T, _D), lambda t, te: (0, 0)),
                pl.BlockSpec((1, _D, 2 * _FF), lambda t, te: (te[t], 0, 0)),
                pl.BlockSpec((1, _FF, _D), lambda t, te: (te[t], 0, 0)),
            ],
            out_specs=pl.BlockSpec((_TILE, _D), lambda t, te: (t, 0)),
        ),
        out_shape=jax.ShapeDtypeStruct((_NT_PAD, _D), jnp.bfloat16),
    )(te_flat, stok, x_bf, w_gate_up, w_down)

    out = pl.pallas_call(
        _combine_kernel,
        grid=(_T // _TT,),
        in_specs=[
            pl.BlockSpec((_TT, _D), lambda i: (i, 0)),
            pl.BlockSpec((_D, 2 * _SFF), lambda i: (0, 0)),
            pl.BlockSpec((_SFF, _D), lambda i: (0, 0)),
            pl.BlockSpec((_NT_PAD, _D), lambda i: (0, 0)),
            pl.BlockSpec((_TT, _K), lambda i: (i, 0)),
            pl.BlockSpec((_TT, _K), lambda i: (i, 0)),
        ],
        out_specs=pl.BlockSpec((_TT, _D), lambda i: (i, 0)),
        out_shape=jax.ShapeDtypeStruct((_T, _D), jnp.float32),
    )(x_bf, sgu_bf, sd_bf, os, pos2, topk_w)

    return out
